# Initial kernel scaffold; baseline (speedup 1.0000x reference)
import jax
import jax.numpy as jnp
from jax.experimental import pallas as pl


def kernel(r, zi, zj, c):
    raise NotImplementedError("write your pallas kernel here")


# speedup vs baseline: 13.3751x; 13.3751x over previous
import functools

import jax
import jax.numpy as jnp
from jax import lax
from jax.experimental import pallas as pl
from jax.experimental.pallas import tpu as pltpu
from jax.experimental.pallas import tpu_sc as plsc

N_U = 8
N_B = 10
R_CUT = 5.0
INV_R_CUT = 1.0 / R_CUT

NUM_CORES = 2
NUM_SUBCORES = 16
LANES = 16
NW = NUM_CORES * NUM_SUBCORES

CHUNK = 2000


def _sc_body(r_hbm, zi_hbm, zj_hbm, w_hbm, out_hbm, wt_v, r_v, zi_v, zj_v, out_v):
    epw = r_hbm.shape[0] // NW
    n_chunks = epw // CHUNK
    cid = lax.axis_index("c")
    sid = lax.axis_index("s")
    wid = sid * NUM_CORES + cid
    base = wid * epw

    pltpu.sync_copy(w_hbm, wt_v)

    iota8 = lax.iota(jnp.int32, LANES) * N_U

    def chunk_body(ci, carry):
        e0 = base + ci * CHUNK
        pltpu.sync_copy(r_hbm.at[pl.ds(e0, CHUNK)], r_v)
        pltpu.sync_copy(zi_hbm.at[pl.ds(e0, CHUNK)], zi_v)
        pltpu.sync_copy(zj_hbm.at[pl.ds(e0, CHUNK)], zj_v)

        def vec_body(i, c2):
            s = i * LANES
            rr = r_v[pl.ds(s, LANES)]
            zi16 = zi_v[pl.ds(s, LANES)]
            zj16 = zj_v[pl.ds(s, LANES)]
            widx = (zi16 * 4 + zj16) * (N_U * N_B)
            t = rr * INV_R_CUT
            env = (1.0 - t) * (1.0 - t)
            env = jnp.where(rr < R_CUT, env, 0.0)
            t2 = t + t
            s_prev = env
            s_cur = t * env
            accs = [None] * N_U
            for u in range(N_U):
                cv = plsc.load_gather(wt_v, [widx + (u * N_B)])
                accs[u] = cv * s_prev
            for u in range(N_U):
                cv = plsc.load_gather(wt_v, [widx + (u * N_B + 1)])
                accs[u] = accs[u] + cv * s_cur
            for b in range(2, N_B):
                s_next = t2 * s_cur - s_prev
                s_prev = s_cur
                s_cur = s_next
                for u in range(N_U):
                    cv = plsc.load_gather(wt_v, [widx + (u * N_B + b)])
                    accs[u] = accs[u] + cv * s_cur
            oidx = iota8 + (s * N_U)
            for u in range(N_U):
                plsc.store_scatter(out_v, [oidx + u], accs[u])
            return c2

        lax.fori_loop(0, CHUNK // LANES, vec_body, 0)
        pltpu.sync_copy(out_v, out_hbm.at[pl.ds(e0 * N_U, CHUNK * N_U)])
        return carry

    lax.fori_loop(0, n_chunks, chunk_body, 0)


def kernel(r, zi, zj, c):
    e = r.shape[0]
    w = c.reshape(-1)
    mesh = plsc.VectorSubcoreMesh(core_axis_name="c", subcore_axis_name="s")
    call = functools.partial(
        pl.kernel,
        mesh=mesh,
        compiler_params=pltpu.CompilerParams(needs_layout_passes=False),
        out_type=jax.ShapeDtypeStruct((e * N_U,), jnp.float32),
        scratch_types=[
            pltpu.VMEM((16 * N_U * N_B,), jnp.float32),
            pltpu.VMEM((CHUNK,), jnp.float32),
            pltpu.VMEM((CHUNK,), jnp.int32),
            pltpu.VMEM((CHUNK,), jnp.int32),
            pltpu.VMEM((CHUNK * N_U,), jnp.float32),
        ],
    )(_sc_body)
    out_flat = call(r, zi, zj, w)
    return out_flat.reshape(e, N_U)


# speedup vs baseline: 13.9451x; 1.0426x over previous
import functools

import jax
import jax.numpy as jnp
from jax import lax
from jax.experimental import pallas as pl
from jax.experimental.pallas import tpu as pltpu
from jax.experimental.pallas import tpu_sc as plsc

N_U = 8
N_B = 10
R_CUT = 5.0
INV_R_CUT = 1.0 / R_CUT

NUM_CORES = 2
NUM_SUBCORES = 16
LANES = 16
NW = NUM_CORES * NUM_SUBCORES

CHUNK = 2000


def _sc_body(r_hbm, zi_hbm, zj_hbm, w_hbm, out_hbm, wt_v, r_v, zi_v, zj_v,
             out_v, isem0, isem1, osem0, osem1):
    epw = r_hbm.shape[0] // NW
    n_chunks = epw // CHUNK
    cid = lax.axis_index("c")
    sid = lax.axis_index("s")
    wid = sid * NUM_CORES + cid
    base = wid * epw

    in_sems = (isem0, isem1)
    out_sems = (osem0, osem1)

    pltpu.sync_copy(w_hbm, wt_v)

    iota8 = lax.iota(jnp.int32, LANES) * N_U

    def start_in(b, ci):
        e0 = base + ci * CHUNK
        off = b * CHUNK
        pltpu.async_copy(r_hbm.at[pl.ds(e0, CHUNK)],
                         r_v.at[pl.ds(off, CHUNK)], in_sems[b])
        pltpu.async_copy(zi_hbm.at[pl.ds(e0, CHUNK)],
                         zi_v.at[pl.ds(off, CHUNK)], in_sems[b])
        pltpu.async_copy(zj_hbm.at[pl.ds(e0, CHUNK)],
                         zj_v.at[pl.ds(off, CHUNK)], in_sems[b])

    def wait_in(b):
        off = b * CHUNK
        pltpu.make_async_copy(r_hbm.at[pl.ds(base, CHUNK)],
                              r_v.at[pl.ds(off, CHUNK)], in_sems[b]).wait()
        pltpu.make_async_copy(zi_hbm.at[pl.ds(base, CHUNK)],
                              zi_v.at[pl.ds(off, CHUNK)], in_sems[b]).wait()
        pltpu.make_async_copy(zj_hbm.at[pl.ds(base, CHUNK)],
                              zj_v.at[pl.ds(off, CHUNK)], in_sems[b]).wait()

    def start_out(b, ci):
        e0 = base + ci * CHUNK
        pltpu.async_copy(out_v.at[pl.ds(b * CHUNK * N_U, CHUNK * N_U)],
                         out_hbm.at[pl.ds(e0 * N_U, CHUNK * N_U)], out_sems[b])

    def wait_out(b):
        pltpu.make_async_copy(out_v.at[pl.ds(b * CHUNK * N_U, CHUNK * N_U)],
                              out_hbm.at[pl.ds(base * N_U, CHUNK * N_U)],
                              out_sems[b]).wait()

    def compute(b):
        off = b * CHUNK
        ooff = b * CHUNK * N_U

        def vec_body(i, c2):
            s = off + i * LANES
            rr = r_v[pl.ds(s, LANES)]
            zi16 = zi_v[pl.ds(s, LANES)]
            zj16 = zj_v[pl.ds(s, LANES)]
            widx = (zi16 * 4 + zj16) * (N_U * N_B)
            t = rr * INV_R_CUT
            env = (1.0 - t) * (1.0 - t)
            env = jnp.where(rr < R_CUT, env, 0.0)
            t2 = t + t
            s_prev = env
            s_cur = t * env
            accs = [None] * N_U
            for u in range(N_U):
                cv = plsc.load_gather(wt_v, [widx + (u * N_B)])
                accs[u] = cv * s_prev
            for u in range(N_U):
                cv = plsc.load_gather(wt_v, [widx + (u * N_B + 1)])
                accs[u] = accs[u] + cv * s_cur
            for bb in range(2, N_B):
                s_next = t2 * s_cur - s_prev
                s_prev = s_cur
                s_cur = s_next
                for u in range(N_U):
                    cv = plsc.load_gather(wt_v, [widx + (u * N_B + bb)])
                    accs[u] = accs[u] + cv * s_cur
            oidx = iota8 + (ooff + i * LANES * N_U)
            for u in range(N_U):
                plsc.store_scatter(out_v, [oidx + u], accs[u])
            return c2

        lax.fori_loop(0, CHUNK // LANES, vec_body, 0)

    start_in(0, 0)

    def chunk_body(ci, carry):
        def process(b):
            @pl.when(ci + 1 < n_chunks)
            def _():
                start_in(1 - b, ci + 1)

            wait_in(b)

            @pl.when(ci >= 2)
            def _():
                wait_out(b)

            compute(b)
            start_out(b, ci)

        @pl.when(ci % 2 == 0)
        def _():
            process(0)

        @pl.when(ci % 2 == 1)
        def _():
            process(1)

        return carry

    lax.fori_loop(0, n_chunks, chunk_body, 0)
    wait_out(0)
    wait_out(1)


def kernel(r, zi, zj, c):
    e = r.shape[0]
    w = c.reshape(-1)
    mesh = plsc.VectorSubcoreMesh(core_axis_name="c", subcore_axis_name="s")
    call = functools.partial(
        pl.kernel,
        mesh=mesh,
        compiler_params=pltpu.CompilerParams(needs_layout_passes=False),
        out_type=jax.ShapeDtypeStruct((e * N_U,), jnp.float32),
        scratch_types=[
            pltpu.VMEM((16 * N_U * N_B,), jnp.float32),
            pltpu.VMEM((2 * CHUNK,), jnp.float32),
            pltpu.VMEM((2 * CHUNK,), jnp.int32),
            pltpu.VMEM((2 * CHUNK,), jnp.int32),
            pltpu.VMEM((2 * CHUNK * N_U,), jnp.float32),
            pltpu.SemaphoreType.DMA,
            pltpu.SemaphoreType.DMA,
            pltpu.SemaphoreType.DMA,
            pltpu.SemaphoreType.DMA,
        ],
    )(_sc_body)
    out_flat = call(r, zi, zj, w)
    return out_flat.reshape(e, N_U)


# speedup vs baseline: 21.6535x; 1.5528x over previous
import functools

import jax
import jax.numpy as jnp
from jax import lax
from jax.experimental import pallas as pl
from jax.experimental.pallas import tpu as pltpu
from jax.experimental.pallas import tpu_sc as plsc

N_U = 8
N_B = 10
R_CUT = 5.0
INV_R_CUT = 1.0 / R_CUT

NUM_CORES = 2
NUM_SUBCORES = 16
LANES = 16
NW = NUM_CORES * NUM_SUBCORES

CHUNK = 2000
ROW = 81


def _sc_body(r_hbm, zi_hbm, zj_hbm, w_hbm, out_hbm, wt_v, r_v, zi_v, zj_v,
             out_v, isem0, isem1, osem0, osem1):
    epw = r_hbm.shape[0] // NW
    n_chunks = epw // CHUNK
    cid = lax.axis_index("c")
    sid = lax.axis_index("s")
    wid = sid * NUM_CORES + cid
    base = wid * epw

    in_sems = (isem0, isem1)
    out_sems = (osem0, osem1)

    pltpu.sync_copy(w_hbm, wt_v)

    iota8 = lax.iota(jnp.int32, LANES) * N_U

    def start_in(b, ci):
        e0 = base + ci * CHUNK
        off = b * CHUNK
        pltpu.async_copy(r_hbm.at[pl.ds(e0, CHUNK)],
                         r_v.at[pl.ds(off, CHUNK)], in_sems[b])
        pltpu.async_copy(zi_hbm.at[pl.ds(e0, CHUNK)],
                         zi_v.at[pl.ds(off, CHUNK)], in_sems[b])
        pltpu.async_copy(zj_hbm.at[pl.ds(e0, CHUNK)],
                         zj_v.at[pl.ds(off, CHUNK)], in_sems[b])

    def wait_in(b):
        off = b * CHUNK
        pltpu.make_async_copy(r_hbm.at[pl.ds(base, CHUNK)],
                              r_v.at[pl.ds(off, CHUNK)], in_sems[b]).wait()
        pltpu.make_async_copy(zi_hbm.at[pl.ds(base, CHUNK)],
                              zi_v.at[pl.ds(off, CHUNK)], in_sems[b]).wait()
        pltpu.make_async_copy(zj_hbm.at[pl.ds(base, CHUNK)],
                              zj_v.at[pl.ds(off, CHUNK)], in_sems[b]).wait()

    def start_out(b, ci):
        e0 = base + ci * CHUNK
        pltpu.async_copy(out_v.at[pl.ds(b * CHUNK * N_U, CHUNK * N_U)],
                         out_hbm.at[pl.ds(e0 * N_U, CHUNK * N_U)], out_sems[b])

    def wait_out(b):
        pltpu.make_async_copy(out_v.at[pl.ds(b * CHUNK * N_U, CHUNK * N_U)],
                              out_hbm.at[pl.ds(base * N_U, CHUNK * N_U)],
                              out_sems[b]).wait()

    def compute(b):
        off = b * CHUNK
        ooff = b * CHUNK * N_U

        def vec_body(i, c2):
            s = off + i * LANES
            rr = r_v[pl.ds(s, LANES)]
            zi16 = zi_v[pl.ds(s, LANES)]
            zj16 = zj_v[pl.ds(s, LANES)]
            widx = (zi16 * 4 + zj16) * ROW
            t = rr * INV_R_CUT
            env = (1.0 - t) * (1.0 - t)
            env = jnp.where(rr < R_CUT, env, 0.0)
            t2 = t + t
            s_prev = env
            s_cur = t * env
            accs = [None] * N_U
            for u in range(N_U):
                cv = plsc.load_gather(wt_v, [widx + (u * N_B)])
                accs[u] = cv * s_prev
            for u in range(N_U):
                cv = plsc.load_gather(wt_v, [widx + (u * N_B + 1)])
                accs[u] = accs[u] + cv * s_cur
            for bb in range(2, N_B):
                s_next = t2 * s_cur - s_prev
                s_prev = s_cur
                s_cur = s_next
                for u in range(N_U):
                    cv = plsc.load_gather(wt_v, [widx + (u * N_B + bb)])
                    accs[u] = accs[u] + cv * s_cur
            oidx = iota8 + (ooff + i * LANES * N_U)
            for u in range(N_U):
                plsc.store_scatter(out_v, [oidx + u], accs[u])
            return c2

        lax.fori_loop(0, CHUNK // LANES, vec_body, 0)

    start_in(0, 0)

    def chunk_body(ci, carry):
        def process(b):
            @pl.when(ci + 1 < n_chunks)
            def _():
                start_in(1 - b, ci + 1)

            wait_in(b)

            @pl.when(ci >= 2)
            def _():
                wait_out(b)

            compute(b)
            start_out(b, ci)

        @pl.when(ci % 2 == 0)
        def _():
            process(0)

        @pl.when(ci % 2 == 1)
        def _():
            process(1)

        return carry

    lax.fori_loop(0, n_chunks, chunk_body, 0)
    wait_out(0)
    wait_out(1)


def kernel(r, zi, zj, c):
    e = r.shape[0]
    w = c.reshape(16, N_U * N_B)
    w = jnp.pad(w, ((0, 0), (0, ROW - N_U * N_B))).reshape(-1)
    mesh = plsc.VectorSubcoreMesh(core_axis_name="c", subcore_axis_name="s")
    call = functools.partial(
        pl.kernel,
        mesh=mesh,
        compiler_params=pltpu.CompilerParams(needs_layout_passes=False),
        out_type=jax.ShapeDtypeStruct((e * N_U,), jnp.float32),
        scratch_types=[
            pltpu.VMEM((16 * ROW,), jnp.float32),
            pltpu.VMEM((2 * CHUNK,), jnp.float32),
            pltpu.VMEM((2 * CHUNK,), jnp.int32),
            pltpu.VMEM((2 * CHUNK,), jnp.int32),
            pltpu.VMEM((2 * CHUNK * N_U,), jnp.float32),
            pltpu.SemaphoreType.DMA,
            pltpu.SemaphoreType.DMA,
            pltpu.SemaphoreType.DMA,
            pltpu.SemaphoreType.DMA,
        ],
    )(_sc_body)
    out_flat = call(r, zi, zj, w)
    return out_flat.reshape(e, N_U)
